# baseline (speedup 1.0000x reference)
import functools

import jax
import jax.numpy as jnp
from jax import lax
from jax.experimental import pallas as pl
from jax.experimental.pallas import tpu as pltpu
from jax.experimental.pallas import tpu_sc as plsc

_N = 640000
_NC = 2
_NS = 16
_NW = _NC * _NS
_CH = _N // _NW


@functools.partial(
    pl.kernel,
    mesh=plsc.VectorSubcoreMesh(core_axis_name="c", subcore_axis_name="s"),
    out_type=jax.ShapeDtypeStruct((_N,), jnp.int32),
    scratch_types=[pltpu.VMEM((_CH,), jnp.int32)],
)
def _sc_copy(src_hbm, out_hbm, buf):
    wid = lax.axis_index("s") * _NC + lax.axis_index("c")
    base = wid * _CH
    pltpu.sync_copy(src_hbm.at[pl.ds(base, _CH)], buf)
    pltpu.sync_copy(buf, out_hbm.at[pl.ds(base, _CH)])


def kernel(node_features, edges, layer_lengths, W1, b1, W2, b2):
    flat = edges.reshape(_N)
    return _sc_copy(flat).reshape(edges.shape)


# speedup vs baseline: 2.2584x; 2.2584x over previous
import jax
import jax.numpy as jnp
from jax.experimental import pallas as pl
from jax.experimental.pallas import tpu as pltpu

_N = 640000
_N_CHUNKS = 2
_CH = _N // _N_CHUNKS


def _dma_pipe_kernel(src, dst, *scratch):
    bufs = scratch[:_N_CHUNKS]
    in_sems, out_sems = scratch[_N_CHUNKS], scratch[_N_CHUNKS + 1]

    def in_copy(i):
        return pltpu.make_async_copy(
            src.at[pl.ds(i * _CH, _CH)], bufs[i], in_sems.at[i])

    def out_copy(i):
        return pltpu.make_async_copy(
            bufs[i], dst.at[pl.ds(i * _CH, _CH)], out_sems.at[i])

    for i in range(_N_CHUNKS):
        in_copy(i).start()
    for i in range(_N_CHUNKS):
        in_copy(i).wait()
        out_copy(i).start()
    for i in range(_N_CHUNKS):
        out_copy(i).wait()


def kernel(node_features, edges, layer_lengths, W1, b1, W2, b2):
    flat = edges.reshape(_N)
    out = pl.pallas_call(
        _dma_pipe_kernel,
        in_specs=[pl.BlockSpec(memory_space=pl.ANY)],
        out_specs=pl.BlockSpec(memory_space=pl.ANY),
        out_shape=jax.ShapeDtypeStruct(flat.shape, flat.dtype),
        scratch_shapes=(
            [pltpu.VMEM((_CH,), jnp.int32) for _ in range(_N_CHUNKS)]
            + [pltpu.SemaphoreType.DMA((_N_CHUNKS,)),
               pltpu.SemaphoreType.DMA((_N_CHUNKS,))]
        ),
    )(flat)
    return out.reshape(edges.shape)
